# baseline (speedup 1.0000x reference)
import functools

import jax
import jax.numpy as jnp
from jax.experimental import pallas as pl
from jax.experimental.pallas import tpu as pltpu

F0, T0 = 64, 128
S0 = F0 * T0
PD0 = (F0 + 4) * T0
F1, T1 = 32, 64
PR1 = 128
S1 = F1 * PR1
PD1 = (F1 + 4) * PR1
PC1 = S1 + 2 * PR1
F2, T2 = 16, 32
C = 64
CONV_D = F2 * C


def _fused_kernel(nb, x_ref, w0_ref, h0_ref, w1s_ref, h1_ref,
                  w1b_ref, b1_ref, w2_ref, b2_ref, selA_ref, selB_ref,
                  o_ref,
                  pad0, col0, y0all, rmv2, po0, act0p, pad1, col1, y1bn,
                  rmv1, po1, act1p, xT, hbuf):
    np_ = nb // 2
    nw = np_ * PC1 - 2 * PR1

    pad0[...] = jnp.zeros_like(pad0)
    pad1[...] = jnp.zeros_like(pad1)

    for r in range(F0):
        pad0[pl.ds(0, nb), pl.ds((r + 2) * T0, T0)] = \
            x_ref[:, 0, r, :].astype(jnp.bfloat16)

    lane0 = jax.lax.broadcasted_iota(jnp.int32, (8, S0), 1) % T0
    for kh in range(3):
        for kw in range(3):
            t = kh * 3 + kw
            src = pad0[:, pl.ds(kh * T0 + kw - 1 + T0, S0)]
            if kw == 0:
                src = jnp.where(lane0 == 0, jnp.bfloat16(0), src)
            elif kw == 2:
                src = jnp.where(lane0 == T0 - 1, jnp.bfloat16(0), src)
            col0[pl.ds(t * 8, 8), :] = src

    y0all[...] = jnp.dot(w0_ref[...], col0[...],
                         preferred_element_type=jnp.float32).astype(jnp.bfloat16)

    for i in range(F1):
        rm = jnp.maximum(y0all[:, pl.ds((2 * i) * T0, T0)],
                         y0all[:, pl.ds((2 * i + 1) * T0, T0)])
        rmv2[pl.ds((i // 2) * nb * C, nb * C), pl.ds((i % 2) * T0, T0)] = rm
    po0[...] = jnp.dot(rmv2[...], selA_ref[...],
                       preferred_element_type=jnp.float32).astype(jnp.bfloat16)
    act0p[...] = jnp.maximum(
        jnp.maximum(po0[:, pl.ds(0, 128)], po0[:, pl.ds(128, 128)])
        + h0_ref[...], 0.0)
    for i in range(F1):
        ip, q = i // 2, i % 2
        blk = act0p[pl.ds(ip * nb * C, nb * C), pl.ds(q * T1, T1)]
        pad1[:, pl.ds((i + 2) * PR1, T1)] = blk[:np_ * C, :]
        pad1[:, pl.ds((i + 2) * PR1 + T1, T1)] = blk[np_ * C:, :]

    lane1 = jax.lax.broadcasted_iota(jnp.int32, (C, PC1), 1) % T1
    for kw in range(3):
        for bp in range(np_):
            src = pad1[pl.ds(bp * C, C), pl.ds(kw - 1 + PR1, PC1)]
            if kw == 0:
                src = jnp.where(lane1 == 0, jnp.bfloat16(0), src)
            elif kw == 2:
                src = jnp.where(lane1 == T1 - 1, jnp.bfloat16(0), src)
            col1[pl.ds(kw * C, C), pl.ds(bp * PC1, PC1)] = src
    y1bn[...] = (
        jnp.dot(w1s_ref[:, pl.ds(0, 192)], col1[:, pl.ds(0, nw)],
                preferred_element_type=jnp.float32)
        + jnp.dot(w1s_ref[:, pl.ds(192, 192)], col1[:, pl.ds(PR1, nw)],
                  preferred_element_type=jnp.float32)
        + jnp.dot(w1s_ref[:, pl.ds(384, 192)], col1[:, pl.ds(2 * PR1, nw)],
                  preferred_element_type=jnp.float32)).astype(jnp.bfloat16)

    for bp in range(np_):
        for i in range(F2):
            rm = jnp.maximum(
                y1bn[:, pl.ds(bp * PC1 + (2 * i) * PR1, PR1)],
                y1bn[:, pl.ds(bp * PC1 + (2 * i + 1) * PR1, PR1)])
            rmv1[pl.ds(bp * 8 * C + (i // 2) * C, C),
                 pl.ds((i % 2) * PR1, PR1)] = rm
    po1[...] = jnp.dot(rmv1[...], selB_ref[...],
                       preferred_element_type=jnp.float32).astype(jnp.bfloat16)
    act1p[...] = jnp.maximum(
        jnp.maximum(po1[:, pl.ds(0, 128)], po1[:, pl.ds(128, 128)])
        + h1_ref[...], 0.0)
    for bp in range(np_):
        for i in range(F2):
            ip, q = i // 2, i % 2
            xT[pl.ds(i * C, C), pl.ds(2 * bp * T2, 2 * T2)] = \
                act1p[pl.ds(bp * 8 * C + ip * C, C), pl.ds(q * 2 * T2, 2 * T2)]

    xfc = jnp.transpose(xT[...], (1, 0))
    o_ref[:, pl.ds(0, CONV_D)] = xfc.astype(jnp.float32)
    xw = jax.lax.dot_general(
        xT[...], w1b_ref[...], (((0,), (0,)), ((), ())),
        preferred_element_type=jnp.float32)
    hbuf[...] = jnp.maximum(xw + b1_ref[...], 0.0).astype(jnp.bfloat16)
    y = jnp.maximum(
        jnp.dot(hbuf[...], w2_ref[...], preferred_element_type=jnp.float32)
        + b2_ref[...], 0.0)
    o_ref[:, pl.ds(CONV_D, 1024)] = y


def _build_sel(block):
    half = block // 2
    w = jax.lax.broadcasted_iota(jnp.int32, (256, 256), 0)
    j = jax.lax.broadcasted_iota(jnp.int32, (256, 256), 1)
    par, jj = j // 128, j % 128
    g, t = jj // half, jj % half
    sel = w == g * block + 2 * t + par
    return jnp.where(sel, 1.0, 0.0).astype(jnp.float32)


@jax.jit
def kernel(x, conv0_w, conv0_scale, conv0_shift,
           conv1_w, conv1_scale, conv1_shift, w1, b1, w2, b2):
    B = x.shape[0]
    nb = 2 if B % 2 == 0 else 1
    rows = nb * T2

    w0m = jnp.transpose(conv0_w, (3, 0, 1, 2)).reshape(C, 9)
    w0s = w0m * conv0_scale.reshape(C, 1)
    h0 = conv0_shift.reshape(C, 1)
    w1m = jnp.transpose(conv1_w, (3, 0, 1, 2)).reshape(C, 9 * C)
    w1s = w1m * conv1_scale.reshape(C, 1)
    h1 = conv1_shift.reshape(C, 1)

    w0blk = jnp.zeros((nb * C, 9 * 8), jnp.float32)
    for b in range(nb):
        p, bp = b % 2, b // 2
        r0 = p * (nb // 2) * C + bp * C
        w0blk = w0blk.at[r0:r0 + C, b::8].set(w0s)
    h0big = jnp.tile(h0, (16 * nb, 1)).astype(jnp.bfloat16)
    h1big = jnp.tile(h1, (4 * nb, 1)).astype(jnp.bfloat16)

    selA = _build_sel(T0)
    selB = _build_sel(T1)

    bf = jnp.bfloat16
    w0blk = w0blk.astype(bf)
    w1s_b = w1s.astype(bf)
    w1b = w1.astype(bf)
    w2_b = w2.astype(bf)
    selA = selA.astype(bf)
    selB = selB.astype(bf)

    const = lambda i: (0, 0)
    grid = (B // nb,)
    out = pl.pallas_call(
        functools.partial(_fused_kernel, nb),
        out_shape=jax.ShapeDtypeStruct((B * T2, 2048), jnp.float32),
        grid=grid,
        in_specs=[
            pl.BlockSpec((nb, 1, F0, T0), lambda i: (i, 0, 0, 0)),
            pl.BlockSpec((nb * C, 9 * 8), const),
            pl.BlockSpec((16 * nb * C, 1), const),
            pl.BlockSpec((C, 9 * C), const),
            pl.BlockSpec((4 * nb * C, 1), const),
            pl.BlockSpec((CONV_D, 2048), const),
            pl.BlockSpec((1, 2048), const),
            pl.BlockSpec((2048, 1024), const),
            pl.BlockSpec((1, 1024), const),
            pl.BlockSpec((256, 256), const),
            pl.BlockSpec((256, 256), const),
        ],
        out_specs=pl.BlockSpec((rows, 2048), lambda i: (i, 0)),
        scratch_shapes=[
            pltpu.VMEM((8, PD0), jnp.bfloat16),
            pltpu.VMEM((9 * 8, S0), jnp.bfloat16),
            pltpu.VMEM((nb * C, S0), jnp.bfloat16),
            pltpu.VMEM((16 * nb * C, 256), jnp.bfloat16),
            pltpu.VMEM((16 * nb * C, 256), jnp.bfloat16),
            pltpu.VMEM((16 * nb * C, 128), jnp.bfloat16),
            pltpu.VMEM((nb * C // 2, PD1), jnp.bfloat16),
            pltpu.VMEM((3 * C, (nb // 2) * PC1), jnp.bfloat16),
            pltpu.VMEM((C, (nb // 2) * PC1 - 2 * PR1), jnp.bfloat16),
            pltpu.VMEM((4 * nb * C, 256), jnp.bfloat16),
            pltpu.VMEM((4 * nb * C, 256), jnp.bfloat16),
            pltpu.VMEM((4 * nb * C, 128), jnp.bfloat16),
            pltpu.VMEM((CONV_D, rows), jnp.bfloat16),
            pltpu.VMEM((rows, 2048), jnp.bfloat16),
        ],
        compiler_params=pltpu.CompilerParams(
            dimension_semantics=("parallel",)),
    )(x, w0blk, h0big, w1s_b, h1big, w1b, b1.reshape(1, 2048), w2_b,
      b2.reshape(1, 1024), selA, selB)
    return out.reshape(B, T2, 2048)


# speedup vs baseline: 1.0409x; 1.0409x over previous
import functools

import jax
import jax.numpy as jnp
from jax.experimental import pallas as pl
from jax.experimental.pallas import tpu as pltpu

F0, T0 = 64, 128
S0 = F0 * T0
PD0 = (F0 + 4) * T0
F1, T1 = 32, 64
PR1 = 128
S1 = F1 * PR1
PD1 = (F1 + 4) * PR1
PC1 = S1 + 2 * PR1
F2, T2 = 16, 32
C = 64
CONV_D = F2 * C


def _fused_kernel(nb, x_ref, w0_ref, h0_ref, w1s_ref, h1_ref,
                  w1b_ref, b1_ref, w2_ref, b2_ref, selA_ref, selB_ref,
                  o_ref,
                  pad0, col0, y0all, rmv2, po0, act0p, pad1, col1, y1bn,
                  rmv1, po1, act1p, xT, hbuf):
    np_ = nb // 2
    nw = np_ * PC1 - 2 * PR1

    pad0[...] = jnp.zeros_like(pad0)
    pad1[...] = jnp.zeros_like(pad1)

    for r in range(F0):
        pad0[pl.ds(0, nb), pl.ds((r + 2) * T0, T0)] = \
            x_ref[:, 0, r, :].astype(jnp.bfloat16)

    lane0 = jax.lax.broadcasted_iota(jnp.int32, (8, S0), 1) % T0
    for kh in range(3):
        for kw in range(3):
            t = kh * 3 + kw
            src = pad0[:, pl.ds(kh * T0 + kw - 1 + T0, S0)]
            if kw == 0:
                src = jnp.where(lane0 == 0, jnp.bfloat16(0), src)
            elif kw == 2:
                src = jnp.where(lane0 == T0 - 1, jnp.bfloat16(0), src)
            col0[pl.ds(t * 8, 8), :] = src

    y0all[...] = jnp.dot(w0_ref[...], col0[...],
                         preferred_element_type=jnp.float32).astype(jnp.bfloat16)

    for i in range(F1):
        rm = jnp.maximum(y0all[:, pl.ds((2 * i) * T0, T0)],
                         y0all[:, pl.ds((2 * i + 1) * T0, T0)])
        rmv2[pl.ds((i // 2) * nb * C, nb * C), pl.ds((i % 2) * T0, T0)] = rm
    po0[...] = jnp.dot(rmv2[...], selA_ref[...],
                       preferred_element_type=jnp.float32).astype(jnp.bfloat16)
    act0p[...] = jnp.maximum(
        jnp.maximum(po0[:, pl.ds(0, 128)], po0[:, pl.ds(128, 128)])
        + h0_ref[...], 0.0)
    for i in range(F1):
        ip, q = i // 2, i % 2
        blk = act0p[pl.ds(ip * nb * C, nb * C), pl.ds(q * T1, T1)]
        pad1[:, pl.ds((i + 2) * PR1, T1)] = blk[:np_ * C, :]
        pad1[:, pl.ds((i + 2) * PR1 + T1, T1)] = blk[np_ * C:, :]

    lane1 = jax.lax.broadcasted_iota(jnp.int32, (C, PC1), 1) % T1
    for kw in range(3):
        for bp in range(np_):
            src = pad1[pl.ds(bp * C, C), pl.ds(kw - 1 + PR1, PC1)]
            if kw == 0:
                src = jnp.where(lane1 == 0, jnp.bfloat16(0), src)
            elif kw == 2:
                src = jnp.where(lane1 == T1 - 1, jnp.bfloat16(0), src)
            col1[pl.ds(kw * C, C), pl.ds(bp * PC1, PC1)] = src
    y1bn[...] = (
        jnp.dot(w1s_ref[:, pl.ds(0, 192)], col1[:, pl.ds(0, nw)],
                preferred_element_type=jnp.float32)
        + jnp.dot(w1s_ref[:, pl.ds(192, 192)], col1[:, pl.ds(PR1, nw)],
                  preferred_element_type=jnp.float32)
        + jnp.dot(w1s_ref[:, pl.ds(384, 192)], col1[:, pl.ds(2 * PR1, nw)],
                  preferred_element_type=jnp.float32)).astype(jnp.bfloat16)

    for bp in range(np_):
        for i in range(F2):
            rm = jnp.maximum(
                y1bn[:, pl.ds(bp * PC1 + (2 * i) * PR1, PR1)],
                y1bn[:, pl.ds(bp * PC1 + (2 * i + 1) * PR1, PR1)])
            rmv1[pl.ds(bp * 8 * C + (i // 2) * C, C),
                 pl.ds((i % 2) * PR1, PR1)] = rm
    po1[...] = jnp.dot(rmv1[...], selB_ref[...],
                       preferred_element_type=jnp.float32).astype(jnp.bfloat16)
    act1p[...] = jnp.maximum(
        jnp.maximum(po1[:, pl.ds(0, 128)], po1[:, pl.ds(128, 128)])
        + h1_ref[...], 0.0)
    for bp in range(np_):
        for i in range(F2):
            ip, q = i // 2, i % 2
            xT[pl.ds(i * C, C), pl.ds(2 * bp * T2, 2 * T2)] = \
                act1p[pl.ds(bp * 8 * C + ip * C, C), pl.ds(q * 2 * T2, 2 * T2)]

    xfc = jnp.transpose(xT[...], (1, 0))
    o_ref[:, pl.ds(0, CONV_D)] = xfc.astype(jnp.float32)
    xw = jax.lax.dot_general(
        xT[...], w1b_ref[...], (((0,), (0,)), ((), ())),
        preferred_element_type=jnp.float32)
    hbuf[...] = jnp.maximum(xw + b1_ref[...], 0.0).astype(jnp.bfloat16)
    y = jnp.maximum(
        jnp.dot(hbuf[...], w2_ref[...], preferred_element_type=jnp.float32)
        + b2_ref[...], 0.0)
    o_ref[:, pl.ds(CONV_D, 1024)] = y


def _build_sel(block):
    half = block // 2
    w = jax.lax.broadcasted_iota(jnp.int32, (256, 256), 0)
    j = jax.lax.broadcasted_iota(jnp.int32, (256, 256), 1)
    par, jj = j // 128, j % 128
    g, t = jj // half, jj % half
    sel = w == g * block + 2 * t + par
    return jnp.where(sel, 1.0, 0.0).astype(jnp.float32)


@jax.jit
def kernel(x, conv0_w, conv0_scale, conv0_shift,
           conv1_w, conv1_scale, conv1_shift, w1, b1, w2, b2):
    B = x.shape[0]
    nb = 4 if B % 4 == 0 else 1
    rows = nb * T2

    w0m = jnp.transpose(conv0_w, (3, 0, 1, 2)).reshape(C, 9)
    w0s = w0m * conv0_scale.reshape(C, 1)
    h0 = conv0_shift.reshape(C, 1)
    w1m = jnp.transpose(conv1_w, (3, 0, 1, 2)).reshape(C, 9 * C)
    w1s = w1m * conv1_scale.reshape(C, 1)
    h1 = conv1_shift.reshape(C, 1)

    w0blk = jnp.zeros((nb * C, 9 * 8), jnp.float32)
    for b in range(nb):
        p, bp = b % 2, b // 2
        r0 = p * (nb // 2) * C + bp * C
        w0blk = w0blk.at[r0:r0 + C, b::8].set(w0s)
    h0big = jnp.tile(h0, (16 * nb, 1)).astype(jnp.bfloat16)
    h1big = jnp.tile(h1, (4 * nb, 1)).astype(jnp.bfloat16)

    selA = _build_sel(T0)
    selB = _build_sel(T1)

    bf = jnp.bfloat16
    w0blk = w0blk.astype(bf)
    w1s_b = w1s.astype(bf)
    w1b = w1.astype(bf)
    w2_b = w2.astype(bf)
    selA = selA.astype(bf)
    selB = selB.astype(bf)

    const = lambda i: (0, 0)
    grid = (B // nb,)
    out = pl.pallas_call(
        functools.partial(_fused_kernel, nb),
        out_shape=jax.ShapeDtypeStruct((B * T2, 2048), jnp.float32),
        grid=grid,
        in_specs=[
            pl.BlockSpec((nb, 1, F0, T0), lambda i: (i, 0, 0, 0)),
            pl.BlockSpec((nb * C, 9 * 8), const),
            pl.BlockSpec((16 * nb * C, 1), const),
            pl.BlockSpec((C, 9 * C), const),
            pl.BlockSpec((4 * nb * C, 1), const),
            pl.BlockSpec((CONV_D, 2048), const),
            pl.BlockSpec((1, 2048), const),
            pl.BlockSpec((2048, 1024), const),
            pl.BlockSpec((1, 1024), const),
            pl.BlockSpec((256, 256), const),
            pl.BlockSpec((256, 256), const),
        ],
        out_specs=pl.BlockSpec((rows, 2048), lambda i: (i, 0)),
        scratch_shapes=[
            pltpu.VMEM((8, PD0), jnp.bfloat16),
            pltpu.VMEM((9 * 8, S0), jnp.bfloat16),
            pltpu.VMEM((nb * C, S0), jnp.bfloat16),
            pltpu.VMEM((16 * nb * C, 256), jnp.bfloat16),
            pltpu.VMEM((16 * nb * C, 256), jnp.bfloat16),
            pltpu.VMEM((16 * nb * C, 128), jnp.bfloat16),
            pltpu.VMEM((nb * C // 2, PD1), jnp.bfloat16),
            pltpu.VMEM((3 * C, (nb // 2) * PC1), jnp.bfloat16),
            pltpu.VMEM((C, (nb // 2) * PC1 - 2 * PR1), jnp.bfloat16),
            pltpu.VMEM((4 * nb * C, 256), jnp.bfloat16),
            pltpu.VMEM((4 * nb * C, 256), jnp.bfloat16),
            pltpu.VMEM((4 * nb * C, 128), jnp.bfloat16),
            pltpu.VMEM((CONV_D, rows), jnp.bfloat16),
            pltpu.VMEM((rows, 2048), jnp.bfloat16),
        ],
        compiler_params=pltpu.CompilerParams(
            dimension_semantics=("parallel",)),
    )(x, w0blk, h0big, w1s_b, h1big, w1b, b1.reshape(1, 2048), w2_b,
      b2.reshape(1, 1024), selA, selB)
    return out.reshape(B, T2, 2048)
